# baseline (speedup 1.0000x reference)
import jax
import jax.numpy as jnp
from jax.experimental import pallas as pl
from jax.experimental.pallas import tpu as pltpu

N = 4096
IN_F = 256
HID = 128
HEADS = 4
NEG = 0.2

BLK = 512

BF = jnp.bfloat16


def _gat_kernel(adj_ref, x_ref, w_ref, a_ref, wout_ref, aout_ref,
                out_ref,
                whext_s, ones_s, es_s, fs_s, edt_s, fdt_s, whmean_s,
                adjb_s, whoext_s, eso_s, fso_s, edot_s, fdot_s, ocsum_s):
    EXTW = 2 * HID
    ph = pl.program_id(0)
    i = pl.program_id(1)
    row0 = i * BLK

    @pl.when((ph == 0) & (i == 0))
    def _prologue():
        x = x_ref[...]
        ext_parts, s_parts, d_parts, mean_parts = [], [], [], []
        for h in range(HEADS):
            wh = jnp.dot(x, w_ref[:, h * HID:(h + 1) * HID],
                         preferred_element_type=jnp.float32)
            s_parts.append(jnp.dot(wh, a_ref[:HID, h:h + 1],
                                   preferred_element_type=jnp.float32))
            d_parts.append(jnp.dot(wh, a_ref[HID:, h:h + 1],
                                   preferred_element_type=jnp.float32))
            mean_parts.append(jnp.mean(wh, axis=0, keepdims=True))
            ext_parts.append(wh.astype(BF))
        whext_s[...] = jnp.concatenate(ext_parts, axis=1)
        ones_s[...] = jnp.ones_like(ones_s)
        whmean_s[...] = jnp.concatenate(mean_parts, axis=1)
        s = jnp.concatenate(s_parts, axis=1)
        d = jnp.concatenate(d_parts, axis=1)
        es_s[...] = jnp.exp(s).astype(BF)
        fs_s[...] = jnp.exp(NEG * s).astype(BF)
        edt_s[...] = jnp.exp(d).astype(BF).T
        fdt_s[...] = jnp.exp(NEG * d).astype(BF).T

    @pl.when(ph == 0)
    def _layer1():
        adj = adj_ref[...].astype(BF)
        adjb_s[pl.ds(row0, BLK), :] = adj.astype(jnp.int8)
        h_parts = []
        for hd in range(HEADS):
            a = es_s[pl.ds(row0, BLK), hd:hd + 1] * edt_s[hd:hd + 1, :]
            b = fs_s[pl.ds(row0, BLK), hd:hd + 1] * fdt_s[hd:hd + 1, :]
            p = jnp.maximum(a, b) * adj
            agg = jnp.dot(p, whext_s[:, hd * HID:(hd + 1) * HID],
                          preferred_element_type=jnp.float32)
            denom = jnp.dot(p, ones_s[...],
                            preferred_element_type=jnp.float32)[:, :1]
            out = jnp.where(denom > 0, agg / denom,
                            whmean_s[:, hd * HID:(hd + 1) * HID])
            out = jnp.where(out > 0, out, jnp.exp(out) - 1.0)
            h_parts.append(out)
        hblk = jnp.concatenate(h_parts, axis=1)
        who = jnp.dot(hblk, wout_ref[...], preferred_element_type=jnp.float32)
        whoext_s[pl.ds(row0, BLK), :] = who.astype(BF)
        so = jnp.dot(who, aout_ref[:HID, :], preferred_element_type=jnp.float32)
        do = jnp.dot(who, aout_ref[HID:, :], preferred_element_type=jnp.float32)
        eso_s[:, pl.ds(row0, BLK)] = jnp.exp(so).astype(BF).T
        fso_s[:, pl.ds(row0, BLK)] = jnp.exp(NEG * so).astype(BF).T
        edot_s[:, pl.ds(row0, BLK)] = jnp.exp(do).astype(BF).T
        fdot_s[:, pl.ds(row0, BLK)] = jnp.exp(NEG * do).astype(BF).T

        @pl.when(i == 0)
        def _():
            ocsum_s[...] = jnp.zeros_like(ocsum_s)
        ocsum_s[...] += jnp.sum(who, axis=0, keepdims=True)

    @pl.when(ph == 1)
    def _layer2():
        adj = adjb_s[pl.ds(row0, BLK), :].astype(BF)
        a = eso_s[:, pl.ds(row0, BLK)].T * edot_s[...]
        b = fso_s[:, pl.ds(row0, BLK)].T * fdot_s[...]
        p = jnp.maximum(a, b) * adj
        agg = jnp.dot(p, whoext_s[...], preferred_element_type=jnp.float32)
        denom = jnp.dot(p, ones_s[...],
                        preferred_element_type=jnp.float32)[:, :1]
        whomean = ocsum_s[...] * (1.0 / N)
        out = jnp.where(denom > 0, agg / denom, whomean)
        out = jnp.where(out > 0, out, jnp.exp(out) - 1.0)
        m2 = jnp.max(out, axis=1, keepdims=True)
        zz = out - m2
        out_ref[...] = zz - jnp.log(jnp.sum(jnp.exp(zz), axis=1,
                                            keepdims=True))


def kernel(x, adj, W0, a0, W1, a1, W2, a2, W3, a3, W_out, a_out):
    f32 = jnp.float32
    W_cat = jnp.concatenate([W0, W1, W2, W3], axis=1)
    a_cat = jnp.concatenate([a0, a1, a2, a3], axis=1)

    FH = HEADS * HID
    EXTW = 2 * HID
    out = pl.pallas_call(
        _gat_kernel,
        grid=(2, N // BLK),
        in_specs=[
            pl.BlockSpec((BLK, N), lambda p, i: (i * (1 - p), 0)),
            pl.BlockSpec((N, IN_F), lambda p, i: (0, 0)),
            pl.BlockSpec((IN_F, FH), lambda p, i: (0, 0)),
            pl.BlockSpec((2 * HID, HEADS), lambda p, i: (0, 0)),
            pl.BlockSpec((FH, HID), lambda p, i: (0, 0)),
            pl.BlockSpec((2 * HID, 1), lambda p, i: (0, 0)),
        ],
        out_specs=pl.BlockSpec((BLK, HID), lambda p, i: (i, 0)),
        out_shape=jax.ShapeDtypeStruct((N, HID), jnp.float32),
        scratch_shapes=[
            pltpu.VMEM((N, HEADS * HID), BF),
            pltpu.VMEM((N, 128), BF),
            pltpu.VMEM((N, HEADS), BF),
            pltpu.VMEM((N, HEADS), BF),
            pltpu.VMEM((HEADS, N), BF),
            pltpu.VMEM((HEADS, N), BF),
            pltpu.VMEM((1, FH), f32),
            pltpu.VMEM((N, N), jnp.int8),
            pltpu.VMEM((N, HID), BF),
            pltpu.VMEM((1, N), BF),
            pltpu.VMEM((1, N), BF),
            pltpu.VMEM((1, N), BF),
            pltpu.VMEM((1, N), BF),
            pltpu.VMEM((1, HID), f32),
        ],
    )(adj, x.astype(BF), W_cat.astype(BF), a_cat, W_out, a_out)
    return out


# speedup vs baseline: 1.5300x; 1.5300x over previous
import jax
import jax.numpy as jnp
from jax.experimental import pallas as pl
from jax.experimental.pallas import tpu as pltpu

N = 4096
IN_F = 256
HID = 128
HEADS = 4
NEG = 0.2

BLK_B = 512
BLK_C = 512

BF = jnp.bfloat16


def _attn1_kernel(adj_ref, x_ref, w_ref, a_ref, wout_ref, aout_ref,
                  adjb_ref, whoext_ref, eso_ref, fso_ref, edot_ref, fdot_ref,
                  csum_ref,
                  whext_s, es_s, fs_s, edt_s, fdt_s, whmean_s):
    EXTW = 2 * HID

    @pl.when(pl.program_id(0) == 0)
    def _prologue():
        x = x_ref[...].astype(BF)
        ext_parts, s_parts, d_parts, mean_parts = [], [], [], []
        for h in range(HEADS):
            wh = jnp.dot(x, w_ref[:, h * HID:(h + 1) * HID].astype(BF),
                         preferred_element_type=jnp.float32)
            s_parts.append(jnp.dot(wh, a_ref[:HID, h:h + 1],
                                   preferred_element_type=jnp.float32))
            d_parts.append(jnp.dot(wh, a_ref[HID:, h:h + 1],
                                   preferred_element_type=jnp.float32))
            mean_parts.append(jnp.mean(wh, axis=0, keepdims=True))
            ext_parts.append(wh.astype(BF))
            ext_parts.append(jnp.ones((N, 1), BF))
            ext_parts.append(jnp.zeros((N, HID - 1), BF))
        whext_s[...] = jnp.concatenate(ext_parts, axis=1)
        whmean_s[...] = jnp.concatenate(mean_parts, axis=1)
        s = jnp.concatenate(s_parts, axis=1)
        d = jnp.concatenate(d_parts, axis=1)
        es_s[...] = jnp.exp(s).astype(BF)
        fs_s[...] = jnp.exp(NEG * s).astype(BF)
        edt_s[...] = jnp.exp(d).astype(BF).T
        fdt_s[...] = jnp.exp(NEG * d).astype(BF).T

    i = pl.program_id(0)
    adj = adj_ref[...].astype(BF)
    adjb_ref[...] = adj.astype(jnp.int8)
    row0 = i * BLK_B
    h_parts = []
    for hd in range(HEADS):
        a = es_s[pl.ds(row0, BLK_B), hd:hd + 1] * edt_s[hd:hd + 1, :]
        b = fs_s[pl.ds(row0, BLK_B), hd:hd + 1] * fdt_s[hd:hd + 1, :]
        p = jnp.maximum(a, b) * adj
        agg_ext = jnp.dot(p, whext_s[:, hd * EXTW:(hd + 1) * EXTW],
                          preferred_element_type=jnp.float32)
        agg = agg_ext[:, :HID]
        denom = agg_ext[:, HID:HID + 1]
        out = jnp.where(denom > 0, agg / denom,
                        whmean_s[:, hd * HID:(hd + 1) * HID])
        out = jnp.where(out > 0, out, jnp.exp(out) - 1.0)
        h_parts.append(out)
    hblk = jnp.concatenate(h_parts, axis=1)
    who = jnp.dot(hblk, wout_ref[...], preferred_element_type=jnp.float32)
    whoext_ref[...] = jnp.concatenate(
        [who.astype(BF), jnp.ones((BLK_B, 1), BF),
         jnp.zeros((BLK_B, HID - 1), BF)], axis=1)
    so = jnp.dot(who, aout_ref[:HID, :], preferred_element_type=jnp.float32)
    do = jnp.dot(who, aout_ref[HID:, :], preferred_element_type=jnp.float32)
    eso_ref[...] = jnp.exp(so).astype(BF)
    fso_ref[...] = jnp.exp(NEG * so).astype(BF)
    edot_ref[...] = jnp.exp(do).astype(BF).T
    fdot_ref[...] = jnp.exp(NEG * do).astype(BF).T

    @pl.when(i == 0)
    def _():
        csum_ref[...] = jnp.zeros_like(csum_ref)
    csum_ref[...] += jnp.sum(who, axis=0, keepdims=True)


def _attn2_kernel(adj_ref, whoext_ref, eso_ref, fso_ref, edot_ref, fdot_ref,
                  csum_in_ref, out_ref):
    adj = adj_ref[...].astype(BF)
    a = eso_ref[...] * edot_ref[...]
    b = fso_ref[...] * fdot_ref[...]
    p = jnp.maximum(a, b) * adj
    agg_ext = jnp.dot(p, whoext_ref[...], preferred_element_type=jnp.float32)
    agg = agg_ext[:, :HID]
    denom = agg_ext[:, HID:HID + 1]
    whomean = csum_in_ref[...] * (1.0 / N)
    out = jnp.where(denom > 0, agg / denom, whomean)
    out = jnp.where(out > 0, out, jnp.exp(out) - 1.0)
    m2 = jnp.max(out, axis=1, keepdims=True)
    zz = out - m2
    out_ref[...] = zz - jnp.log(jnp.sum(jnp.exp(zz), axis=1, keepdims=True))


def kernel(x, adj, W0, a0, W1, a1, W2, a2, W3, a3, W_out, a_out):
    f32 = jnp.float32
    W_cat = jnp.concatenate([W0, W1, W2, W3], axis=1)
    a_cat = jnp.concatenate([a0, a1, a2, a3], axis=1)

    FH = HEADS * HID
    EXTW = 2 * HID
    adj_bf, whoext, eso, fso, edot, fdot, who_csum = pl.pallas_call(
        _attn1_kernel,
        grid=(N // BLK_B,),
        in_specs=[
            pl.BlockSpec((BLK_B, N), lambda i: (i, 0)),
            pl.BlockSpec((N, IN_F), lambda i: (0, 0)),
            pl.BlockSpec((IN_F, FH), lambda i: (0, 0)),
            pl.BlockSpec((2 * HID, HEADS), lambda i: (0, 0)),
            pl.BlockSpec((FH, HID), lambda i: (0, 0)),
            pl.BlockSpec((2 * HID, 1), lambda i: (0, 0)),
        ],
        out_specs=[
            pl.BlockSpec((BLK_B, N), lambda i: (i, 0)),
            pl.BlockSpec((BLK_B, EXTW), lambda i: (i, 0)),
            pl.BlockSpec((BLK_B, 1), lambda i: (i, 0)),
            pl.BlockSpec((BLK_B, 1), lambda i: (i, 0)),
            pl.BlockSpec((1, BLK_B), lambda i: (0, i)),
            pl.BlockSpec((1, BLK_B), lambda i: (0, i)),
            pl.BlockSpec((1, HID), lambda i: (0, 0)),
        ],
        out_shape=[
            jax.ShapeDtypeStruct((N, N), jnp.int8),
            jax.ShapeDtypeStruct((N, EXTW), BF),
            jax.ShapeDtypeStruct((N, 1), BF),
            jax.ShapeDtypeStruct((N, 1), BF),
            jax.ShapeDtypeStruct((1, N), BF),
            jax.ShapeDtypeStruct((1, N), BF),
            jax.ShapeDtypeStruct((1, HID), f32),
        ],
        scratch_shapes=[
            pltpu.VMEM((N, HEADS * EXTW), BF),
            pltpu.VMEM((N, HEADS), BF),
            pltpu.VMEM((N, HEADS), BF),
            pltpu.VMEM((HEADS, N), BF),
            pltpu.VMEM((HEADS, N), BF),
            pltpu.VMEM((1, FH), f32),
        ],
    )(adj, x, W_cat, a_cat, W_out, a_out)

    out = pl.pallas_call(
        _attn2_kernel,
        grid=(N // BLK_C,),
        in_specs=[
            pl.BlockSpec((BLK_C, N), lambda i: (i, 0)),
            pl.BlockSpec((N, EXTW), lambda i: (0, 0)),
            pl.BlockSpec((BLK_C, 1), lambda i: (i, 0)),
            pl.BlockSpec((BLK_C, 1), lambda i: (i, 0)),
            pl.BlockSpec((1, N), lambda i: (0, 0)),
            pl.BlockSpec((1, N), lambda i: (0, 0)),
            pl.BlockSpec((1, HID), lambda i: (0, 0)),
        ],
        out_specs=pl.BlockSpec((BLK_C, HID), lambda i: (i, 0)),
        out_shape=jax.ShapeDtypeStruct((N, HID), jnp.float32),
    )(adj_bf, whoext, eso, fso, edot, fdot, who_csum)
    return out
